# baseline (speedup 1.0000x reference)
import functools

import jax
import jax.numpy as jnp
from jax import lax
from jax.experimental import pallas as pl
from jax.experimental.pallas import tpu as pltpu
from jax.experimental.pallas import tpu_sc as plsc

_NUM_ROWS = 32
_NUM_COLS = 32
_EMBED_DIM = 768
_LANES = 16
_CHUNKS = _EMBED_DIM // _LANES
_GROUP = 8
_NGROUPS = _NUM_COLS // _GROUP

_mesh = plsc.VectorSubcoreMesh(core_axis_name="c", subcore_axis_name="s")


@functools.partial(
    pl.kernel,
    mesh=_mesh,
    out_type=jax.ShapeDtypeStruct((_NUM_ROWS * _NUM_COLS, _EMBED_DIM), jnp.float32),
    scratch_types=[
        pltpu.VMEM((_NUM_ROWS,), jnp.int32),
        pltpu.VMEM((1 + _NUM_COLS, _EMBED_DIM), jnp.float32),
        pltpu.VMEM((_NUM_COLS, _EMBED_DIM), jnp.float32),
        pltpu.SemaphoreType.DMA,
        pltpu.SemaphoreType.DMA,
        pltpu.SemaphoreType.DMA,
    ],
)
def _pos2d(wrow_hbm, wcol_hbm, out_hbm, idx_v, wc_v, out_v, gsem, csem, osem):
    num_cores = 2
    wid = lax.axis_index("s") * num_cores + lax.axis_index("c")
    row = jnp.full((_LANES,), wid + 1, jnp.int32)
    idx_v[pl.ds(0, _LANES)] = row
    idx_v[pl.ds(_LANES, _LANES)] = row
    gather = pltpu.async_copy(wrow_hbm.at[idx_v], out_v, gsem)
    wcol_cp = pltpu.async_copy(wcol_hbm, wc_v, csem)
    gather.wait()
    wcol_cp.wait()

    for g in range(_NGROUPS):
        def col_body(c, _):
            for j in range(_CHUNKS):
                sl = pl.ds(j * _LANES, _LANES)
                plsc.addupdate(out_v.at[c, sl], wc_v[c + 1, sl])
            return 0

        lax.fori_loop(g * _GROUP, (g + 1) * _GROUP, col_body, 0)
        pltpu.async_copy(
            out_v.at[pl.ds(g * _GROUP, _GROUP)],
            out_hbm.at[pl.ds(wid * _NUM_COLS + g * _GROUP, _GROUP)],
            osem,
        )
    pltpu.make_async_copy(
        out_v, out_hbm.at[pl.ds(wid * _NUM_COLS, _NUM_COLS)], osem
    ).wait()


def kernel(input, W_row, W_col):
    del input
    out = _pos2d(W_row, W_col)
    return out.reshape(1, _NUM_ROWS * _NUM_COLS, _EMBED_DIM)
